# speedup vs baseline: 2.9032x; 2.9032x over previous
import jax
import jax.numpy as jnp
from jax.experimental import pallas as pl
from jax.experimental.pallas import tpu as pltpu

_SEQ_TILE = 512


def _add_kernel(x_ref, emb_ref, out_ref):
    out_ref[0] = x_ref[0] + emb_ref[...]


def kernel(x, emb_weight):
    batch, seq_len, d_model = x.shape
    ts = _SEQ_TILE
    grid = (seq_len // ts, batch)
    return pl.pallas_call(
        _add_kernel,
        grid=grid,
        in_specs=[
            pl.BlockSpec((1, ts, d_model), lambda s, b: (b, s, 0)),
            pl.BlockSpec((ts, d_model), lambda s, b: (s, 0)),
        ],
        out_specs=pl.BlockSpec((1, ts, d_model), lambda s, b: (b, s, 0)),
        out_shape=jax.ShapeDtypeStruct((batch, seq_len, d_model), x.dtype),
    )(x, emb_weight)


# speedup vs baseline: 3.2540x; 1.1208x over previous
import jax
import jax.numpy as jnp
from jax.experimental import pallas as pl
from jax.experimental.pallas import tpu as pltpu

_SEQ_TILE = 1024


def _add_kernel(x_ref, emb_ref, out_ref):
    out_ref[0] = x_ref[0] + emb_ref[...]


def kernel(x, emb_weight):
    batch, seq_len, d_model = x.shape
    ts = _SEQ_TILE
    grid = (seq_len // ts, batch)
    return pl.pallas_call(
        _add_kernel,
        grid=grid,
        in_specs=[
            pl.BlockSpec((1, ts, d_model), lambda s, b: (b, s, 0)),
            pl.BlockSpec((ts, d_model), lambda s, b: (s, 0)),
        ],
        out_specs=pl.BlockSpec((1, ts, d_model), lambda s, b: (b, s, 0)),
        out_shape=jax.ShapeDtypeStruct((batch, seq_len, d_model), x.dtype),
    )(x, emb_weight)


# speedup vs baseline: 3.4582x; 1.0627x over previous
import jax
import jax.numpy as jnp
from jax.experimental import pallas as pl
from jax.experimental.pallas import tpu as pltpu

_SEQ_TILE = 2048


def _add_kernel(x_ref, emb_ref, out_ref):
    out_ref[0] = x_ref[0] + emb_ref[...]


def kernel(x, emb_weight):
    batch, seq_len, d_model = x.shape
    ts = _SEQ_TILE
    grid = (seq_len // ts, batch)
    return pl.pallas_call(
        _add_kernel,
        grid=grid,
        in_specs=[
            pl.BlockSpec((1, ts, d_model), lambda s, b: (b, s, 0)),
            pl.BlockSpec((ts, d_model), lambda s, b: (s, 0)),
        ],
        out_specs=pl.BlockSpec((1, ts, d_model), lambda s, b: (b, s, 0)),
        out_shape=jax.ShapeDtypeStruct((batch, seq_len, d_model), x.dtype),
    )(x, emb_weight)
